# baseline (speedup 1.0000x reference)
import functools

import jax
import jax.numpy as jnp
from jax import lax
from jax.experimental import pallas as pl
from jax.experimental.pallas import tpu as pltpu
from jax.experimental.pallas import tpu_sc as plsc

_EMBED = 64
_SCALE = 8.0
_NC, _NS = 2, 16
_NW = _NC * _NS
_BB = 128


@functools.lru_cache(maxsize=None)
def _make_gather_kernel(n_seq, n_batch):
    n_bc = n_batch // _BB
    n_blocks = n_seq * n_bc
    blocks_per_w = n_blocks // _NW
    n_super = blocks_per_w // 8

    mesh = plsc.VectorSubcoreMesh(core_axis_name="c", subcore_axis_name="s")

    @functools.partial(
        pl.kernel,
        out_type=jax.ShapeDtypeStruct((n_seq, _EMBED, n_batch), jnp.float32),
        mesh=mesh,
        scratch_types=[
            pltpu.VMEM((n_super, 8, _BB), jnp.int32),
            pltpu.VMEM((2, _BB), jnp.int32),
            pltpu.VMEM((2, _BB, 129), jnp.float32),
            pltpu.VMEM((2, _EMBED, _BB), jnp.float32),
            pltpu.SemaphoreType.DMA,
            pltpu.SemaphoreType.DMA,
            pltpu.SemaphoreType.DMA,
            pltpu.SemaphoreType.DMA,
            pltpu.SemaphoreType.DMA,
        ],
        compiler_params=pltpu.CompilerParams(needs_layout_passes=False),
    )
    def k(idx_hbm, tab_hbm, out_hbm, idx_v, idx2_v, in_v, out_v,
          si, sg0, sg1, sw0, sw1):
        cid = lax.axis_index("c")
        sid = lax.axis_index("s")
        wid = sid * _NC + cid
        sb0 = wid * n_super

        for u in range(n_super):
            sb = sb0 + u
            st = sb // n_bc
            bc = sb % n_bc
            pltpu.async_copy(
                idx_hbm.at[pl.ds(st * 8, 8), pl.ds(bc * _BB, _BB)],
                idx_v.at[u],
                si,
            )
        for u in range(n_super):
            pltpu.make_async_copy(
                idx_hbm.at[pl.ds(0, 8), pl.ds(0, _BB)], idx_v.at[u], si
            ).wait()

        sg = (sg0, sg1)
        sw = (sw0, sw1)
        iota = lax.iota(jnp.int32, 16)
        rowv = [iota + bg * 16 for bg in range(8)]

        def block_coords(j):
            sb = sb0 + (j // 8)
            s = (sb // n_bc) * 8 + (j % 8)
            bc = sb % n_bc
            return s, bc

        def issue_gather(j, buf):
            for bg in range(8):
                sl = pl.ds(bg * 16, 16)
                idx2_v[buf, sl] = lax.shift_right_logical(
                    idx_v[j // 8, j % 8, sl], 1
                )
            pltpu.async_copy(
                tab_hbm.at[idx2_v.at[buf]],
                in_v.at[buf, pl.ds(0, _BB), pl.ds(0, _BB)],
                sg[buf],
            )

        def wait_gather(buf):
            pltpu.make_async_copy(
                tab_hbm.at[idx2_v.at[buf]],
                in_v.at[buf, pl.ds(0, _BB), pl.ds(0, _BB)],
                sg[buf],
            ).wait()

        def issue_wb(j, buf):
            s, bc = block_coords(j)
            pltpu.async_copy(
                out_v.at[buf],
                out_hbm.at[s, :, pl.ds(bc * _BB, _BB)],
                sw[buf],
            )

        def wait_wb(buf):
            pltpu.make_async_copy(
                out_v.at[buf],
                out_hbm.at[0, :, pl.ds(0, _BB)],
                sw[buf],
            ).wait()

        def compute(j, buf):
            halfv = []
            for bg in range(8):
                idxv = idx_v[j // 8, j % 8, pl.ds(bg * 16, 16)]
                halfv.append(lax.shift_left(idxv & 1, 6))

            @plsc.parallel_loop(0, _EMBED, unroll=4)
            def _(e):
                for bg in range(8):
                    vals = plsc.load_gather(
                        in_v.at[buf], [rowv[bg], halfv[bg] + e]
                    )
                    out_v[buf, e, pl.ds(bg * 16, 16)] = vals * _SCALE

            return None

        issue_gather(0, 0)

        @pl.loop(0, blocks_per_w, step=2)
        def _(jd):
            for db in range(2):
                j = jd + db

                @pl.when(j + 1 < blocks_per_w)
                def _():
                    issue_gather(j + 1, 1 - db)

                wait_gather(db)

                @pl.when(jd >= 2)
                def _():
                    wait_wb(db)

                compute(j, db)
                issue_wb(j, db)

        wait_wb(0)
        wait_wb(1)

    return k


@jax.jit
def kernel(input, table):
    b, s = input.shape
    vocab, embed = table.shape
    idx_t = input.T
    tab2 = table.reshape(vocab // 2, 2 * embed)
    out_phys = _make_gather_kernel(s, b)(idx_t, tab2)
    return out_phys.transpose(2, 0, 1)
